# baseline (speedup 1.0000x reference)
import jax
import jax.numpy as jnp
from jax.experimental import pallas as pl
from jax.experimental.pallas import tpu as pltpu

B, N, F = 16, 4096, 512
E = 64
HI = 2048


def _body(x_any, out_any, lo_v, hi_v, copy_sem, tail_sem, store_sem):
    ld_lo = pltpu.make_async_copy(x_any.at[:, pl.ds(0, E), :], lo_v, tail_sem)
    ld_hi = pltpu.make_async_copy(x_any.at[:, pl.ds(HI, E), :], hi_v, tail_sem)
    ld_lo.start()
    ld_hi.start()

    copies = []
    for b in range(B):
        cp = pltpu.make_async_copy(
            x_any.at[b], out_any.at[b, pl.ds(0, N), :], copy_sem
        )
        cp.start()
        copies.append(cp)

    ld_lo.wait()
    ld_hi.wait()
    lo_v[...] = 0.5 * (lo_v[...] + hi_v[...])
    st = pltpu.make_async_copy(lo_v, out_any.at[:, pl.ds(N, E), :], store_sem)
    st.start()
    st.wait()
    for cp in copies:
        cp.wait()


def kernel(inputs):
    return pl.pallas_call(
        _body,
        in_specs=[pl.BlockSpec(memory_space=pltpu.MemorySpace.HBM)],
        out_specs=pl.BlockSpec(memory_space=pltpu.MemorySpace.HBM),
        out_shape=jax.ShapeDtypeStruct((B, N + E, F), inputs.dtype),
        scratch_shapes=[
            pltpu.VMEM((B, E, F), jnp.float32),
            pltpu.VMEM((B, E, F), jnp.float32),
            pltpu.SemaphoreType.DMA,
            pltpu.SemaphoreType.DMA,
            pltpu.SemaphoreType.DMA,
        ],
    )(inputs)
